# baseline (speedup 1.0000x reference)
import functools

import jax
import jax.numpy as jnp
from jax import lax
from jax.experimental import pallas as pl
from jax.experimental.pallas import tpu as pltpu
from jax.experimental.pallas import tpu_sc as plsc

_B, _L, _V, _D = 1024, 200, 1_000_000, 64
_DP = 128
_N = _B * _L
_C = 128
_NW = 32
_RPW = _N // _NW
_CPW = _RPW // _C
_BV = 512


def _transpose_pad(table_t):
    grid = (pl.cdiv(_V, _BV),)

    def body(in_ref, out_ref):
        y = in_ref[...].T
        out_ref[...] = jnp.concatenate(
            [y, jnp.zeros((_BV, _DP - _D), jnp.float32)], axis=1
        )

    return pl.pallas_call(
        body,
        grid=grid,
        in_specs=[pl.BlockSpec((_D, _BV), lambda i: (0, i))],
        out_specs=pl.BlockSpec((_BV, _DP), lambda i: (i, 0)),
        out_shape=jax.ShapeDtypeStruct((_V, _DP), jnp.float32),
    )(table_t)


def _make_gather():
    mesh = plsc.VectorSubcoreMesh(core_axis_name="c", subcore_axis_name="s")

    @functools.partial(
        pl.kernel,
        mesh=mesh,
        out_type=jax.ShapeDtypeStruct((_N, _DP), jnp.float32),
        scratch_types=[
            pltpu.VMEM((_RPW,), jnp.int32),
            pltpu.VMEM((_C, _DP), jnp.float32),
            pltpu.SemaphoreType.DMA,
        ],
    )
    def gather_kernel(idx_hbm, table_hbm, out_hbm, idx_v, rows_v, sem):
        wid = lax.axis_index("s") * 2 + lax.axis_index("c")
        rbase = wid * _RPW
        pltpu.sync_copy(idx_hbm.at[pl.ds(rbase, _RPW)], idx_v)

        def body(j, carry):
            idx_slice = idx_v.at[pl.ds(j * _C, _C)]
            pltpu.async_copy(table_hbm.at[idx_slice], rows_v, sem).wait()
            pltpu.sync_copy(rows_v, out_hbm.at[pl.ds(rbase + j * _C, _C)])
            return carry

        lax.fori_loop(0, _CPW, body, 0)

    return gather_kernel


_gather = _make_gather()


def kernel(sequences, offsets, table):
    del offsets
    idx = sequences.reshape(_N).astype(jnp.int32)
    table_p = _transpose_pad(table.T)
    out = _gather(idx, table_p)
    return out[:, :_D].reshape(_B, _L, _D)


# speedup vs baseline: 2.6938x; 2.6938x over previous
import functools

import jax
import jax.numpy as jnp
from jax import lax
from jax.experimental import pallas as pl
from jax.experimental.pallas import tpu as pltpu
from jax.experimental.pallas import tpu_sc as plsc

_B, _L, _V, _D = 1024, 200, 1_000_000, 64
_DP = 128
_N = _B * _L
_C = 128
_NW = 32
_RPW = _N // _NW
_CPW = _RPW // _C
_BV = 4096


def _transpose_pad(table_t):
    grid = (pl.cdiv(_V, _BV),)

    def body(in_ref, out_ref):
        y = in_ref[...].T
        out_ref[...] = jnp.concatenate(
            [y, jnp.zeros((_BV, _DP - _D), jnp.float32)], axis=1
        )

    return pl.pallas_call(
        body,
        grid=grid,
        in_specs=[pl.BlockSpec((_D, _BV), lambda i: (0, i))],
        out_specs=pl.BlockSpec((_BV, _DP), lambda i: (i, 0)),
        out_shape=jax.ShapeDtypeStruct((_V, _DP), jnp.float32),
    )(table_t)


def _make_gather():
    mesh = plsc.VectorSubcoreMesh(core_axis_name="c", subcore_axis_name="s")

    @functools.partial(
        pl.kernel,
        mesh=mesh,
        out_type=jax.ShapeDtypeStruct((_N, _DP), jnp.float32),
        scratch_types=[
            pltpu.VMEM((_RPW,), jnp.int32),
            pltpu.VMEM((_C, _DP), jnp.float32),
            pltpu.SemaphoreType.DMA,
        ],
    )
    def gather_kernel(idx_hbm, table_hbm, out_hbm, idx_v, rows_v, sem):
        wid = lax.axis_index("s") * 2 + lax.axis_index("c")
        rbase = wid * _RPW
        pltpu.sync_copy(idx_hbm.at[pl.ds(rbase, _RPW)], idx_v)

        def body(j, carry):
            idx_slice = idx_v.at[pl.ds(j * _C, _C)]
            pltpu.async_copy(table_hbm.at[idx_slice], rows_v, sem).wait()
            pltpu.sync_copy(rows_v, out_hbm.at[pl.ds(rbase + j * _C, _C)])
            return carry

        lax.fori_loop(0, _CPW, body, 0)

    return gather_kernel


_gather = _make_gather()


def kernel(sequences, offsets, table):
    del offsets
    idx = sequences.reshape(_N).astype(jnp.int32)
    table_p = _transpose_pad(table.T)
    out = _gather(idx, table_p)
    return out[:, :_D].reshape(_B, _L, _D)


# speedup vs baseline: 3.1394x; 1.1654x over previous
import functools

import jax
import jax.numpy as jnp
from jax import lax
from jax.experimental import pallas as pl
from jax.experimental.pallas import tpu as pltpu
from jax.experimental.pallas import tpu_sc as plsc

_B, _L, _V, _D = 1024, 200, 1_000_000, 64
_DP = 128
_N = _B * _L
_C = 128
_NW = 32
_RPW = _N // _NW
_CPW = _RPW // _C
_BV = 8192


def _transpose_pad(table_t):
    grid = (pl.cdiv(_V, _BV),)

    def body(in_ref, out_ref):
        y = in_ref[...].T
        out_ref[...] = jnp.concatenate(
            [y, jnp.zeros((_BV, _DP - _D), jnp.float32)], axis=1
        )

    return pl.pallas_call(
        body,
        grid=grid,
        in_specs=[pl.BlockSpec((_D, _BV), lambda i: (0, i))],
        out_specs=pl.BlockSpec((_BV, _DP), lambda i: (i, 0)),
        out_shape=jax.ShapeDtypeStruct((_V, _DP), jnp.float32),
    )(table_t)


def _make_gather():
    mesh = plsc.VectorSubcoreMesh(core_axis_name="c", subcore_axis_name="s")

    @functools.partial(
        pl.kernel,
        mesh=mesh,
        out_type=jax.ShapeDtypeStruct((_N, _DP), jnp.float32),
        scratch_types=[
            pltpu.VMEM((_RPW,), jnp.int32),
            pltpu.VMEM((_C, _DP), jnp.float32),
            pltpu.SemaphoreType.DMA,
        ],
    )
    def gather_kernel(idx_hbm, table_hbm, out_hbm, idx_v, rows_v, sem):
        wid = lax.axis_index("s") * 2 + lax.axis_index("c")
        rbase = wid * _RPW
        pltpu.sync_copy(idx_hbm.at[pl.ds(rbase, _RPW)], idx_v)

        def body(j, carry):
            idx_slice = idx_v.at[pl.ds(j * _C, _C)]
            pltpu.async_copy(table_hbm.at[idx_slice], rows_v, sem).wait()
            pltpu.sync_copy(rows_v, out_hbm.at[pl.ds(rbase + j * _C, _C)])
            return carry

        lax.fori_loop(0, _CPW, body, 0)

    return gather_kernel


_gather = _make_gather()


def kernel(sequences, offsets, table):
    del offsets
    idx = sequences.reshape(_N).astype(jnp.int32)
    table_p = _transpose_pad(table.T)
    out = _gather(idx, table_p)
    return out[:, :_D].reshape(_B, _L, _D)


# speedup vs baseline: 3.2761x; 1.0435x over previous
import functools

import jax
import jax.numpy as jnp
from jax import lax
from jax.experimental import pallas as pl
from jax.experimental.pallas import tpu as pltpu
from jax.experimental.pallas import tpu_sc as plsc

_B, _L, _V, _D = 1024, 200, 1_000_000, 64
_DP = 128
_N = _B * _L
_C = 128
_NW = 32
_RPW = _N // _NW
_CPW = _RPW // _C
_BV = 16384


def _transpose_pad(table_t):
    grid = (pl.cdiv(_V, _BV),)

    def body(in_ref, out_ref):
        y = in_ref[...].T
        out_ref[...] = jnp.concatenate(
            [y, jnp.zeros((_BV, _DP - _D), jnp.float32)], axis=1
        )

    return pl.pallas_call(
        body,
        grid=grid,
        in_specs=[pl.BlockSpec((_D, _BV), lambda i: (0, i))],
        out_specs=pl.BlockSpec((_BV, _DP), lambda i: (i, 0)),
        out_shape=jax.ShapeDtypeStruct((_V, _DP), jnp.float32),
    )(table_t)


def _make_gather():
    mesh = plsc.VectorSubcoreMesh(core_axis_name="c", subcore_axis_name="s")

    @functools.partial(
        pl.kernel,
        mesh=mesh,
        out_type=jax.ShapeDtypeStruct((_N, _DP), jnp.float32),
        scratch_types=[
            pltpu.VMEM((_RPW,), jnp.int32),
            pltpu.VMEM((_C, _DP), jnp.float32),
            pltpu.SemaphoreType.DMA,
        ],
    )
    def gather_kernel(idx_hbm, table_hbm, out_hbm, idx_v, rows_v, sem):
        wid = lax.axis_index("s") * 2 + lax.axis_index("c")
        rbase = wid * _RPW
        pltpu.sync_copy(idx_hbm.at[pl.ds(rbase, _RPW)], idx_v)

        def body(j, carry):
            idx_slice = idx_v.at[pl.ds(j * _C, _C)]
            pltpu.async_copy(table_hbm.at[idx_slice], rows_v, sem).wait()
            pltpu.sync_copy(rows_v, out_hbm.at[pl.ds(rbase + j * _C, _C)])
            return carry

        lax.fori_loop(0, _CPW, body, 0)

    return gather_kernel


_gather = _make_gather()


def kernel(sequences, offsets, table):
    del offsets
    idx = sequences.reshape(_N).astype(jnp.int32)
    table_p = _transpose_pad(table.T)
    out = _gather(idx, table_p)
    return out[:, :_D].reshape(_B, _L, _D)


# speedup vs baseline: 3.3268x; 1.0155x over previous
import functools

import jax
import jax.numpy as jnp
from jax import lax
from jax.experimental import pallas as pl
from jax.experimental.pallas import tpu as pltpu
from jax.experimental.pallas import tpu_sc as plsc

_B, _L, _V, _D = 1024, 200, 1_000_000, 64
_DP = 128
_N = _B * _L
_C = 128
_NW = 32
_RPW = _N // _NW
_CPW = _RPW // _C
_BV = 32768


def _transpose_pad(table_t):
    grid = (pl.cdiv(_V, _BV),)

    def body(in_ref, out_ref):
        y = in_ref[...].T
        out_ref[...] = jnp.concatenate(
            [y, jnp.zeros((_BV, _DP - _D), jnp.float32)], axis=1
        )

    return pl.pallas_call(
        body,
        grid=grid,
        in_specs=[pl.BlockSpec((_D, _BV), lambda i: (0, i))],
        out_specs=pl.BlockSpec((_BV, _DP), lambda i: (i, 0)),
        out_shape=jax.ShapeDtypeStruct((_V, _DP), jnp.float32),
    )(table_t)


def _make_gather():
    mesh = plsc.VectorSubcoreMesh(core_axis_name="c", subcore_axis_name="s")

    @functools.partial(
        pl.kernel,
        mesh=mesh,
        out_type=jax.ShapeDtypeStruct((_N, _DP), jnp.float32),
        scratch_types=[
            pltpu.VMEM((_RPW,), jnp.int32),
            pltpu.VMEM((_C, _DP), jnp.float32),
            pltpu.SemaphoreType.DMA,
        ],
    )
    def gather_kernel(idx_hbm, table_hbm, out_hbm, idx_v, rows_v, sem):
        wid = lax.axis_index("s") * 2 + lax.axis_index("c")
        rbase = wid * _RPW
        pltpu.sync_copy(idx_hbm.at[pl.ds(rbase, _RPW)], idx_v)

        def body(j, carry):
            idx_slice = idx_v.at[pl.ds(j * _C, _C)]
            pltpu.async_copy(table_hbm.at[idx_slice], rows_v, sem).wait()
            pltpu.sync_copy(rows_v, out_hbm.at[pl.ds(rbase + j * _C, _C)])
            return carry

        lax.fori_loop(0, _CPW, body, 0)

    return gather_kernel


_gather = _make_gather()


def kernel(sequences, offsets, table):
    del offsets
    idx = sequences.reshape(_N).astype(jnp.int32)
    table_p = _transpose_pad(table.T)
    out = _gather(idx, table_p)
    return out[:, :_D].reshape(_B, _L, _D)
